# baseline (speedup 1.0000x reference)
import functools

import jax
import jax.numpy as jnp
from jax import lax
from jax.experimental import pallas as pl
from jax.experimental.pallas import tpu as pltpu
from jax.experimental.pallas import tpu_sc as plsc

B = 256
D = 150528
NW = 32
FPW = D // NW
CF = 112
K = FPW // CF
NBUF = 2
NL = B // 16


def _mixup_sc(x_hbm, idx_hbm, lam_hbm, out_hbm,
              pv, lamv, i0, i1, o0, o1,
              isem0, isem1, osem0, osem1):
    nc = plsc.get_sparse_core_info().num_cores
    wid = lax.axis_index("s") * nc + lax.axis_index("c")
    fbase = wid * FPW

    ibufs = (i0, i1)
    obufs = (o0, o1)
    isems = (isem0, isem1)
    osems = (osem0, osem1)

    pltpu.sync_copy(idx_hbm, pv)
    pltpu.sync_copy(lam_hbm, lamv)
    lam = lamv[...]
    one_minus_lam = 1.0 - lam
    pidx = [pv[pl.ds(l * 16, 16)] for l in range(NL)]

    def in_copy(c, slot):
        return pltpu.make_async_copy(
            x_hbm.at[pl.ds(fbase + c * CF, CF)], ibufs[slot], isems[slot])

    def out_copy(c, slot):
        return pltpu.make_async_copy(
            obufs[slot], out_hbm.at[pl.ds(fbase + c * CF, CF)], osems[slot])

    for b in range(NBUF):
        in_copy(b, b).start()

    def step(j, carry):
        for b in range(NBUF):
            c = j * NBUF + b
            in_copy(c, b).wait()

            @pl.when(c >= NBUF)
            def _drain(c=c, b=b):
                out_copy(c - NBUF, b).wait()

            ibuf, obuf = ibufs[b], obufs[b]

            @plsc.parallel_loop(0, CF, step=1, unroll=8)
            def _combine(f, ibuf=ibuf, obuf=obuf):
                row = jnp.broadcast_to(f, (16,))
                for l in range(NL):
                    sl = pl.ds(l * 16, 16)
                    gathered = plsc.load_gather(ibuf, [row, pidx[l]])
                    obuf[f, sl] = lam * ibuf[f, sl] + one_minus_lam * gathered

            out_copy(c, b).start()

            @pl.when(c + NBUF < K)
            def _prefetch(c=c, b=b):
                in_copy(c + NBUF, b).start()
        return carry

    lax.fori_loop(0, K // NBUF, step, 0)

    for c in range(K - K % NBUF, K):
        b = c % NBUF
        in_copy(c, b).wait()
        out_copy(c - NBUF, b).wait()
        ibuf, obuf = ibufs[b], obufs[b]

        @plsc.parallel_loop(0, CF, step=1, unroll=8)
        def _combine_tail(f, ibuf=ibuf, obuf=obuf):
            row = jnp.broadcast_to(f, (16,))
            for l in range(NL):
                sl = pl.ds(l * 16, 16)
                gathered = plsc.load_gather(ibuf, [row, pidx[l]])
                obuf[f, sl] = lam * ibuf[f, sl] + one_minus_lam * gathered

        out_copy(c, b).start()

    for b in range(NBUF):
        out_copy(b, b).wait()


def kernel(inputs, index, lam):
    xt = jnp.reshape(inputs, (B, D)).T
    idx32 = index.astype(jnp.int32)
    lam16 = jnp.full((16,), lam, jnp.float32)

    run = functools.partial(
        pl.kernel,
        out_type=jax.ShapeDtypeStruct((D, B), jnp.float32),
        mesh=plsc.VectorSubcoreMesh(core_axis_name="c", subcore_axis_name="s"),
        compiler_params=pltpu.CompilerParams(needs_layout_passes=False),
        scratch_types=[
            pltpu.VMEM((B,), jnp.int32),
            pltpu.VMEM((16,), jnp.float32),
            pltpu.VMEM((CF, B), jnp.float32),
            pltpu.VMEM((CF, B), jnp.float32),
            pltpu.VMEM((CF, B), jnp.float32),
            pltpu.VMEM((CF, B), jnp.float32),
            pltpu.SemaphoreType.DMA,
            pltpu.SemaphoreType.DMA,
            pltpu.SemaphoreType.DMA,
            pltpu.SemaphoreType.DMA,
        ],
    )(_mixup_sc)
    out_t = run(xt, idx32, lam16)
    return jnp.reshape(out_t.T, inputs.shape)


# speedup vs baseline: 1.4657x; 1.4657x over previous
import functools

import jax
import jax.numpy as jnp
from jax import lax
from jax.experimental import pallas as pl
from jax.experimental.pallas import tpu as pltpu
from jax.experimental.pallas import tpu_sc as plsc

B = 256
D = 150528
NW = 32
FPW = D // NW
CF = 56
K = FPW // CF
NBUF = 3
NL = B // 16


def _mixup_sc(x_hbm, idx_hbm, lam_hbm, out_hbm,
              pv, lamv, i0, i1, i2, o0, o1, o2,
              isem0, isem1, isem2, osem0, osem1, osem2):
    nc = plsc.get_sparse_core_info().num_cores
    wid = lax.axis_index("s") * nc + lax.axis_index("c")
    fbase = wid * FPW

    ibufs = (i0, i1, i2)
    obufs = (o0, o1, o2)
    isems = (isem0, isem1, isem2)
    osems = (osem0, osem1, osem2)

    pltpu.sync_copy(idx_hbm, pv)
    pltpu.sync_copy(lam_hbm, lamv)
    lam = lamv[...]
    one_minus_lam = 1.0 - lam
    pidx = [pv[pl.ds(l * 16, 16)] for l in range(NL)]

    def in_copy(c, slot):
        return pltpu.make_async_copy(
            x_hbm.at[pl.ds(fbase + c * CF, CF)], ibufs[slot], isems[slot])

    def out_copy(c, slot):
        return pltpu.make_async_copy(
            obufs[slot], out_hbm.at[pl.ds(fbase + c * CF, CF)], osems[slot])

    for b in range(NBUF):
        in_copy(b, b).start()

    def step(j, carry):
        for b in range(NBUF):
            c = j * NBUF + b
            in_copy(c, b).wait()

            @pl.when(c >= NBUF)
            def _drain(c=c, b=b):
                out_copy(c - NBUF, b).wait()

            ibuf, obuf = ibufs[b], obufs[b]

            @plsc.parallel_loop(0, CF, step=1, unroll=4)
            def _combine(f, ibuf=ibuf, obuf=obuf):
                row = jnp.broadcast_to(f, (16,))
                for l in range(NL):
                    sl = pl.ds(l * 16, 16)
                    gathered = plsc.load_gather(ibuf, [row, pidx[l]])
                    obuf[f, sl] = lam * ibuf[f, sl] + one_minus_lam * gathered

            out_copy(c, b).start()

            @pl.when(c + NBUF < K)
            def _prefetch(c=c, b=b):
                in_copy(c + NBUF, b).start()
        return carry

    lax.fori_loop(0, K // NBUF, step, 0)

    for c in range(K - K % NBUF, K):
        b = c % NBUF
        in_copy(c, b).wait()
        out_copy(c - NBUF, b).wait()
        ibuf, obuf = ibufs[b], obufs[b]

        @plsc.parallel_loop(0, CF, step=1, unroll=4)
        def _combine_tail(f, ibuf=ibuf, obuf=obuf):
            row = jnp.broadcast_to(f, (16,))
            for l in range(NL):
                sl = pl.ds(l * 16, 16)
                gathered = plsc.load_gather(ibuf, [row, pidx[l]])
                obuf[f, sl] = lam * ibuf[f, sl] + one_minus_lam * gathered

        out_copy(c, b).start()

    for b in range(NBUF):
        out_copy(b, b).wait()


def kernel(inputs, index, lam):
    xt = jnp.reshape(inputs, (B, D)).T
    idx32 = index.astype(jnp.int32)
    lam16 = jnp.full((16,), lam, jnp.float32)

    run = functools.partial(
        pl.kernel,
        out_type=jax.ShapeDtypeStruct((D, B), jnp.float32),
        mesh=plsc.VectorSubcoreMesh(core_axis_name="c", subcore_axis_name="s"),
        compiler_params=pltpu.CompilerParams(needs_layout_passes=False),
        scratch_types=[
            pltpu.VMEM((B,), jnp.int32),
            pltpu.VMEM((16,), jnp.float32),
            pltpu.VMEM((CF, B), jnp.float32),
            pltpu.VMEM((CF, B), jnp.float32),
            pltpu.VMEM((CF, B), jnp.float32),
            pltpu.VMEM((CF, B), jnp.float32),
            pltpu.VMEM((CF, B), jnp.float32),
            pltpu.VMEM((CF, B), jnp.float32),
            pltpu.SemaphoreType.DMA,
            pltpu.SemaphoreType.DMA,
            pltpu.SemaphoreType.DMA,
            pltpu.SemaphoreType.DMA,
            pltpu.SemaphoreType.DMA,
            pltpu.SemaphoreType.DMA,
        ],
    )(_mixup_sc)
    out_t = run(xt, idx32, lam16)
    return jnp.reshape(out_t.T, inputs.shape)


# speedup vs baseline: 1.5544x; 1.0605x over previous
import functools

import jax
import jax.numpy as jnp
from jax import lax
from jax.experimental import pallas as pl
from jax.experimental.pallas import tpu as pltpu
from jax.experimental.pallas import tpu_sc as plsc

B = 256
D = 150528
NW = 32
FPW = D // NW
CF = 112
K = FPW // CF
NBUF = 2
NL = B // 16


def _mixup_sc(x_hbm, idx_hbm, lam_hbm, out_hbm,
              pv, lamv, i0, i1, o0, o1,
              isem0, isem1, osem0, osem1):
    nc = plsc.get_sparse_core_info().num_cores
    wid = lax.axis_index("s") * nc + lax.axis_index("c")
    fbase = wid * FPW

    ibufs = (i0, i1)
    obufs = (o0, o1)
    isems = (isem0, isem1)
    osems = (osem0, osem1)

    pltpu.sync_copy(idx_hbm, pv)
    pltpu.sync_copy(lam_hbm, lamv)
    lam = lamv[...]
    one_minus_lam = 1.0 - lam
    pidx = [pv[pl.ds(l * 16, 16)] for l in range(NL)]

    def in_copy(c, slot):
        return pltpu.make_async_copy(
            x_hbm.at[pl.ds(fbase + c * CF, CF)], ibufs[slot], isems[slot])

    def out_copy(c, slot):
        return pltpu.make_async_copy(
            obufs[slot], out_hbm.at[pl.ds(fbase + c * CF, CF)], osems[slot])

    for b in range(NBUF):
        in_copy(b, b).start()

    def step(j, carry):
        for b in range(NBUF):
            c = j * NBUF + b
            in_copy(c, b).wait()

            @pl.when(c >= NBUF)
            def _drain(c=c, b=b):
                out_copy(c - NBUF, b).wait()

            ibuf, obuf = ibufs[b], obufs[b]

            @plsc.parallel_loop(0, CF, step=1, unroll=4)
            def _combine(f, ibuf=ibuf, obuf=obuf):
                row = jnp.broadcast_to(f, (16,))
                for l in range(NL):
                    sl = pl.ds(l * 16, 16)
                    gathered = plsc.load_gather(ibuf, [row, pidx[l]])
                    obuf[f, sl] = lam * ibuf[f, sl] + one_minus_lam * gathered

            out_copy(c, b).start()

            @pl.when(c + NBUF < K)
            def _prefetch(c=c, b=b):
                in_copy(c + NBUF, b).start()
        return carry

    lax.fori_loop(0, K // NBUF, step, 0)

    for c in range(K - K % NBUF, K):
        b = c % NBUF
        in_copy(c, b).wait()
        out_copy(c - NBUF, b).wait()
        ibuf, obuf = ibufs[b], obufs[b]

        @plsc.parallel_loop(0, CF, step=1, unroll=4)
        def _combine_tail(f, ibuf=ibuf, obuf=obuf):
            row = jnp.broadcast_to(f, (16,))
            for l in range(NL):
                sl = pl.ds(l * 16, 16)
                gathered = plsc.load_gather(ibuf, [row, pidx[l]])
                obuf[f, sl] = lam * ibuf[f, sl] + one_minus_lam * gathered

        out_copy(c, b).start()

    for b in range(NBUF):
        out_copy(b, b).wait()


def kernel(inputs, index, lam):
    xt = jnp.reshape(inputs, (B, D)).T
    idx32 = index.astype(jnp.int32)
    lam16 = jnp.full((16,), lam, jnp.float32)

    run = functools.partial(
        pl.kernel,
        out_type=jax.ShapeDtypeStruct((D, B), jnp.float32),
        mesh=plsc.VectorSubcoreMesh(core_axis_name="c", subcore_axis_name="s"),
        compiler_params=pltpu.CompilerParams(needs_layout_passes=False),
        scratch_types=[
            pltpu.VMEM((B,), jnp.int32),
            pltpu.VMEM((16,), jnp.float32),
            pltpu.VMEM((CF, B), jnp.float32),
            pltpu.VMEM((CF, B), jnp.float32),
            pltpu.VMEM((CF, B), jnp.float32),
            pltpu.VMEM((CF, B), jnp.float32),
            pltpu.SemaphoreType.DMA,
            pltpu.SemaphoreType.DMA,
            pltpu.SemaphoreType.DMA,
            pltpu.SemaphoreType.DMA,
        ],
    )(_mixup_sc)
    out_t = run(xt, idx32, lam16)
    return jnp.reshape(out_t.T, inputs.shape)
